# baseline (speedup 1.0000x reference)
import functools

import numpy as np

import jax
import jax.numpy as jnp
from jax import lax
from jax.experimental import pallas as pl
from jax.experimental.pallas import tpu as pltpu
from jax.experimental.pallas import tpu_sc as plsc

NC = 2
NS = 16
NW = NC * NS
L = 16

B = 4
R = 2048
C = 2048
TOT = B * R
RPW = TOT // NW
K = 16
H = 8
G = RPW // K
NCHUNK = C // L


def _rev4(x: int) -> int:
    return ((x & 1) << 3) | ((x & 2) << 1) | ((x & 4) >> 1) | ((x & 8) >> 3)


def _make_jpos() -> np.ndarray:
    jpos = np.empty((C,), np.int32)
    q = 0
    for d in range(16):
        for w in range(C // 256):
            for t in range(L):
                jpos[q * L + t] = w * 16 + t + ((_rev4(t) ^ d) << 7)
            q += 1
    return jpos


_JPOS = _make_jpos()


def _sc_body(t_hbm, rp_hbm, cp_hbm, jp_hbm, out_hbm,
             idx_v, cp_v, jp_v, ii_v, rows0, rows1, perm0, perm1, perm2,
             is0, is1, os0, os1, os2, stage_sem):
    wid = lax.axis_index("s") * NC + lax.axis_index("c")
    base = wid * RPW
    i0 = base % R
    boff = (base // R) * R

    pltpu.sync_copy(rp_hbm.at[pl.ds(i0, RPW)], idx_v)
    for t in range(RPW // L):
        sl = pl.ds(t * L, L)
        idx_v[sl] = idx_v[sl] + boff

    rows = (rows0, rows1)
    perms = (perm0, perm1, perm2)
    isems = (is0, is1)
    osems = (os0, os1, os2)
    in_copies = [None, None]
    out_copies = [None, None, None]

    def start_in(g):
        b = g % 2
        ridx = idx_v[pl.ds(g * K, K)]
        in_copies[b] = pltpu.async_copy(t_hbm.at[ridx], rows[b], isems[b])

    def compute_half(rows_b, h, perm_p):
        @plsc.parallel_loop(0, NCHUNK, 1, unroll=1)
        def _chunk(u):
            sl = pl.ds(u * L, L)
            jv = jp_v[sl]
            ii = ii_v[sl]
            for r in range(H):
                ridx_sp = jnp.full((L,), h * H + r, dtype=jnp.int32)
                vals = plsc.load_gather(rows_b, [ridx_sp, ii])
                rloc_sp = jnp.full((L,), r, dtype=jnp.int32)
                plsc.store_scatter(perm_p, [rloc_sp, jv], vals)

    cp_copy = pltpu.async_copy(cp_hbm, cp_v, stage_sem)
    jp_copy = pltpu.async_copy(jp_hbm, jp_v, stage_sem)

    start_in(0)
    start_in(1)

    cp_copy.wait()
    jp_copy.wait()

    @plsc.parallel_loop(0, NCHUNK, 1, unroll=4)
    def _build_ii(u):
        sl = pl.ds(u * L, L)
        ii_v[sl] = plsc.load_gather(cp_v, [jp_v[sl]])

    for g in range(G):
        b = g % 2
        in_copies[b].wait()
        for h in range(2):
            slot = (2 * g + h) % 3
            if out_copies[slot] is not None:
                out_copies[slot].wait()
            compute_half(rows[b], h, perms[slot])
            out_copies[slot] = pltpu.async_copy(
                perms[slot], out_hbm.at[pl.ds(base + g * K + h * H, H)], osems[slot]
            )
        if g + 2 < G:
            start_in(g + 2)
    for slot in range(3):
        if out_copies[slot] is not None:
            out_copies[slot].wait()


@functools.partial(jax.jit, static_argnames=())
def _sc_permute(t_flat, rowperm, colperm):
    mesh = plsc.VectorSubcoreMesh(
        core_axis_name="c", subcore_axis_name="s", num_cores=NC, num_subcores=NS
    )
    jpos = jnp.asarray(_JPOS)
    return pl.kernel(
        _sc_body,
        out_type=jax.ShapeDtypeStruct((TOT, C), jnp.float32),
        mesh=mesh,
        scratch_types=[
            pltpu.VMEM((RPW,), jnp.int32),
            pltpu.VMEM((C,), jnp.int32),
            pltpu.VMEM((C,), jnp.int32),
            pltpu.VMEM((C,), jnp.int32),
            pltpu.VMEM((K, C), jnp.float32),
            pltpu.VMEM((K, C), jnp.float32),
            pltpu.VMEM((H, C), jnp.float32),
            pltpu.VMEM((H, C), jnp.float32),
            pltpu.VMEM((H, C), jnp.float32),
            pltpu.SemaphoreType.DMA,
            pltpu.SemaphoreType.DMA,
            pltpu.SemaphoreType.DMA,
            pltpu.SemaphoreType.DMA,
            pltpu.SemaphoreType.DMA,
            pltpu.SemaphoreType.DMA,
        ],
        compiler_params=pltpu.CompilerParams(needs_layout_passes=False),
    )(t_flat, rowperm, colperm, jpos)


def kernel(tensor, rowperm, colperm):
    t_flat = tensor.reshape(TOT, C)
    out = _sc_permute(t_flat, rowperm, colperm)
    return out.reshape(B, R, C)


# speedup vs baseline: 1.0048x; 1.0048x over previous
import functools

import numpy as np

import jax
import jax.numpy as jnp
from jax import lax
from jax.experimental import pallas as pl
from jax.experimental.pallas import tpu as pltpu
from jax.experimental.pallas import tpu_sc as plsc

NC = 2
NS = 16
NW = NC * NS
L = 16

B = 4
R = 2048
C = 2048
TOT = B * R
RPW = TOT // NW
K = 16
H = 8
G = RPW // K
NCHUNK = C // L


def _rev4(x: int) -> int:
    return ((x & 1) << 3) | ((x & 2) << 1) | ((x & 4) >> 1) | ((x & 8) >> 3)


def _make_jpos() -> np.ndarray:
    jpos = np.empty((C,), np.int32)
    q = 0
    for d in range(16):
        for w in range(C // 256):
            for t in range(L):
                jpos[q * L + t] = w * 16 + t + ((_rev4(t) ^ d) << 7)
            q += 1
    return jpos


_JPOS = _make_jpos()


def _sc_body(t_hbm, rp_hbm, cp_hbm, jp_hbm, out_hbm,
             idx_v, cp_v, jp_v, ii_v, rows0, rows1, perm0, perm1,
             is0, is1, os0, os1, stage_sem):
    wid = lax.axis_index("s") * NC + lax.axis_index("c")
    base = wid * RPW
    i0 = base % R
    boff = (base // R) * R

    pltpu.sync_copy(rp_hbm.at[pl.ds(i0, RPW)], idx_v)
    for t in range(RPW // L):
        sl = pl.ds(t * L, L)
        idx_v[sl] = idx_v[sl] + boff

    rows = (rows0, rows1)
    perms = (perm0, perm1)
    isems = (is0, is1)
    osems = (os0, os1)
    in_copies = [None, None]
    out_copies = [None, None]

    def start_in(g):
        b = g % 2
        ridx = idx_v[pl.ds(g * K, K)]
        in_copies[b] = pltpu.async_copy(t_hbm.at[ridx], rows[b], isems[b])

    def compute_half(rows_b, h, perm_p):
        @plsc.parallel_loop(0, NCHUNK, 1, unroll=1)
        def _chunk(u):
            sl = pl.ds(u * L, L)
            jv = jp_v[sl]
            ii = ii_v[sl]
            for r in range(H):
                ridx_sp = jnp.full((L,), h * H + r, dtype=jnp.int32)
                vals = plsc.load_gather(rows_b, [ridx_sp, ii])
                rloc_sp = jnp.full((L,), r, dtype=jnp.int32)
                plsc.store_scatter(perm_p, [rloc_sp, jv], vals)

    cp_copy = pltpu.async_copy(cp_hbm, cp_v, stage_sem)
    jp_copy = pltpu.async_copy(jp_hbm, jp_v, stage_sem)

    start_in(0)
    start_in(1)

    cp_copy.wait()
    jp_copy.wait()

    @plsc.parallel_loop(0, NCHUNK, 1, unroll=4)
    def _build_ii(u):
        sl = pl.ds(u * L, L)
        ii_v[sl] = plsc.load_gather(cp_v, [jp_v[sl]])

    for g in range(G):
        b = g % 2
        in_copies[b].wait()
        for h in range(2):
            if out_copies[h] is not None:
                out_copies[h].wait()
            compute_half(rows[b], h, perms[h])
            out_copies[h] = pltpu.async_copy(
                perms[h], out_hbm.at[pl.ds(base + g * K + h * H, H)], osems[h]
            )
        if g + 2 < G:
            start_in(g + 2)
    for h in range(2):
        out_copies[h].wait()


@functools.partial(jax.jit, static_argnames=())
def _sc_permute(t_flat, rowperm, colperm):
    mesh = plsc.VectorSubcoreMesh(
        core_axis_name="c", subcore_axis_name="s", num_cores=NC, num_subcores=NS
    )
    jpos = jnp.asarray(_JPOS)
    return pl.kernel(
        _sc_body,
        out_type=jax.ShapeDtypeStruct((TOT, C), jnp.float32),
        mesh=mesh,
        scratch_types=[
            pltpu.VMEM((RPW,), jnp.int32),
            pltpu.VMEM((C,), jnp.int32),
            pltpu.VMEM((C,), jnp.int32),
            pltpu.VMEM((C,), jnp.int32),
            pltpu.VMEM((K, C), jnp.float32),
            pltpu.VMEM((K, C), jnp.float32),
            pltpu.VMEM((H, C), jnp.float32),
            pltpu.VMEM((H, C), jnp.float32),
            pltpu.SemaphoreType.DMA,
            pltpu.SemaphoreType.DMA,
            pltpu.SemaphoreType.DMA,
            pltpu.SemaphoreType.DMA,
            pltpu.SemaphoreType.DMA,
        ],
        compiler_params=pltpu.CompilerParams(needs_layout_passes=False),
    )(t_flat, rowperm, colperm, jpos)


def kernel(tensor, rowperm, colperm):
    t_flat = tensor.reshape(TOT, C)
    out = _sc_permute(t_flat, rowperm, colperm)
    return out.reshape(B, R, C)
